# baseline (speedup 1.0000x reference)
import functools

import jax
import jax.numpy as jnp
from jax import lax
from jax.experimental import pallas as pl
from jax.experimental.pallas import tpu as pltpu
from jax.experimental.pallas import tpu_sc as plsc


def _tc_body(x_ref, pos_ref, out_ref):
    out_ref[...] = x_ref[...] + pos_ref[...][None, :, :]


def _tc_add(x, pos_embedding, nb):
    B, S, D = x.shape
    BS = 4096
    grid = (S // BS, nb)
    return pl.pallas_call(
        _tc_body,
        grid=grid,
        in_specs=[
            pl.BlockSpec((1, BS, D), lambda s, b: (b, s, 0)),
            pl.BlockSpec((BS, D), lambda s, b: (s, 0)),
        ],
        out_specs=pl.BlockSpec((1, BS, D), lambda s, b: (b, s, 0)),
        out_shape=jax.ShapeDtypeStruct((nb, S, D), x.dtype),
        compiler_params=pltpu.CompilerParams(
            dimension_semantics=("arbitrary", "arbitrary"),
            vmem_limit_bytes=120 * 1024 * 1024,
        ),
    )(x, pos_embedding[:S])


_NC = 2
_NS = 16
_NW = _NC * _NS
_LANES = 16


def _sc_add_v2(xf, posf, B, S, D):
    RW = S // _NW
    CH = 16
    NE = CH * D
    NCH = RW // CH
    U = 8
    assert B == 4 and NCH % 2 == 0

    mesh = plsc.VectorSubcoreMesh(core_axis_name="c", subcore_axis_name="s")
    f32 = jnp.float32

    @functools.partial(
        pl.kernel,
        mesh=mesh,
        out_type=jax.ShapeDtypeStruct((B * S * D,), f32),
        scratch_types=(
            [pltpu.VMEM((NE,), f32) for _ in range(4)]
            + [pltpu.VMEM((NE,), f32) for _ in range(2)]
            + [pltpu.SemaphoreType.DMA] * 4
            + [pltpu.SemaphoreType.DMA] * 4
            + [pltpu.SemaphoreType.DMA] * 2
        ),
    )
    def k(x_hbm, pos_hbm, out_hbm, *refs):
        x_bufs = refs[0:4]
        pos_bufs = refs[4:6]
        in_sems = refs[6:10]
        out_sems = refs[10:14]
        pos_sems = refs[14:16]

        wid = lax.axis_index("s") * _NC + lax.axis_index("c")
        row_base = wid * RW

        def xoff(c, b):
            return (b * S + row_base + c * CH) * D

        def poff(c):
            return (row_base + c * CH) * D

        def start_in(c, b, slot):
            pltpu.async_copy(
                x_hbm.at[pl.ds(xoff(c, b), NE)], x_bufs[slot], in_sems[slot]
            )

        def wait_in(c, b, slot):
            pltpu.make_async_copy(
                x_hbm.at[pl.ds(xoff(c, b), NE)], x_bufs[slot], in_sems[slot]
            ).wait()

        def start_out(c, b, slot):
            pltpu.async_copy(
                x_bufs[slot], out_hbm.at[pl.ds(xoff(c, b), NE)], out_sems[slot]
            )

        def wait_out(c, b, slot):
            pltpu.make_async_copy(
                x_bufs[slot], out_hbm.at[pl.ds(xoff(c, b), NE)], out_sems[slot]
            ).wait()

        def start_pos(c, slot):
            pltpu.async_copy(
                pos_hbm.at[pl.ds(poff(c), NE)], pos_bufs[slot], pos_sems[slot]
            )

        def wait_pos(c, slot):
            pltpu.make_async_copy(
                pos_hbm.at[pl.ds(poff(c), NE)], pos_bufs[slot], pos_sems[slot]
            ).wait()

        def add_chunk(xb, pb):
            def add_body(j, _):
                base = j * (_LANES * U)
                for u in range(U):
                    o = base + u * _LANES
                    xb[pl.ds(o, _LANES)] = (
                        xb[pl.ds(o, _LANES)] + pb[pl.ds(o, _LANES)]
                    )
                return 0

            lax.fori_loop(0, NE // (_LANES * U), add_body, 0)

        start_pos(0, 0)
        start_in(0, 0, 0)
        start_in(0, 1, 1)

        def m_body(m, _):
            for uu in range(2):
                c = 2 * m + uu
                pslot = uu
                for b in range(4):
                    kk = 4 * uu + b
                    s2 = (b + 2) % 4
                    c_wait = c if b >= 2 else c - 1
                    c_next = c + 1 if b >= 2 else c
                    if kk < 2:
                        @pl.when(m > 0)
                        def _():
                            wait_out(c_wait, s2, s2)
                        start_in(c_next, s2, s2)
                    elif kk < 6:
                        wait_out(c_wait, s2, s2)
                        start_in(c_next, s2, s2)
                    else:
                        wait_out(c_wait, s2, s2)
                        @pl.when(m < NCH // 2 - 1)
                        def _():
                            start_in(c_next, s2, s2)
                    if b == 2:
                        if uu == 0:
                            start_pos(c + 1, 1)
                        else:
                            @pl.when(m < NCH // 2 - 1)
                            def _():
                                start_pos(c + 1, 0)
                    if b == 0:
                        wait_pos(c, pslot)
                    wait_in(c, b, b)
                    add_chunk(x_bufs[b], pos_bufs[pslot])
                    start_out(c, b, b)
            return 0

        lax.fori_loop(0, NCH // 2, m_body, 0)
        for b in (2, 3):
            wait_out(NCH - 1, b, b)

    return k(xf, posf)


def _sc_add_v3(xf, posf, B, S, D):
    RW = S // _NW
    CH = 32
    NE = CH * D
    NCH = RW // CH
    NK = NCH * B
    U = 8

    mesh = plsc.VectorSubcoreMesh(core_axis_name="c", subcore_axis_name="s")
    f32 = jnp.float32

    @functools.partial(
        pl.kernel,
        mesh=mesh,
        out_type=jax.ShapeDtypeStruct((B * S * D,), f32),
        scratch_types=(
            [pltpu.VMEM((NE,), f32) for _ in range(2)]
            + [pltpu.VMEM((NE,), f32)]
            + [pltpu.SemaphoreType.DMA] * 2
            + [pltpu.SemaphoreType.DMA] * 2
        ),
    )
    def k(x_hbm, pos_hbm, out_hbm, *refs):
        x_bufs = refs[0:2]
        pos_v = refs[2]
        in_sems = refs[3:5]
        out_sems = refs[5:7]

        wid = lax.axis_index("s") * _NC + lax.axis_index("c")
        row_base = wid * RW

        def xoff(kk):
            return ((kk % B) * S + row_base + (kk // B) * CH) * D

        def start_in(kk, slot):
            pltpu.async_copy(
                x_hbm.at[pl.ds(xoff(kk), NE)], x_bufs[slot], in_sems[slot]
            )

        def wait_in(kk, slot):
            pltpu.make_async_copy(
                x_hbm.at[pl.ds(xoff(kk), NE)], x_bufs[slot], in_sems[slot]
            ).wait()

        def start_out(kk, slot):
            pltpu.async_copy(
                x_bufs[slot], out_hbm.at[pl.ds(xoff(kk), NE)], out_sems[slot]
            )

        def wait_out(kk, slot):
            pltpu.make_async_copy(
                x_bufs[slot], out_hbm.at[pl.ds(xoff(kk), NE)], out_sems[slot]
            ).wait()

        def add_chunk(xb):
            def add_body(j, _):
                base = j * (_LANES * U)
                for u in range(U):
                    o = base + u * _LANES
                    xb[pl.ds(o, _LANES)] = (
                        xb[pl.ds(o, _LANES)] + pos_v[pl.ds(o, _LANES)]
                    )
                return 0

            lax.fori_loop(0, NE // (_LANES * U), add_body, 0)

        start_in(0, 0)

        def m_body(m, _):
            for j in range(2):
                kk = 2 * m + j
                slot = j
                nslot = 1 - j

                @pl.when(kk >= 1)
                def _():
                    wait_out(kk - 1, nslot)

                @pl.when(kk + 1 < NK)
                def _():
                    start_in(kk + 1, nslot)

                @pl.when(kk % B == 0)
                def _():
                    pltpu.sync_copy(
                        pos_hbm.at[
                            pl.ds((row_base + (kk // B) * CH) * D, NE)
                        ],
                        pos_v,
                    )

                wait_in(kk, slot)
                add_chunk(x_bufs[slot])
                start_out(kk, slot)
            return 0

        lax.fori_loop(0, NK // 2, m_body, 0)
        wait_out(NK - 1, 1)

    return k(xf, posf)


def _sc_add(xf, posf, b0, nb, S, D):
    rows = nb * S
    rows_per_w = rows // _NW
    CH = 32
    NE = CH * D
    NCH = rows_per_w // CH
    U = 8

    mesh = plsc.VectorSubcoreMesh(core_axis_name="c", subcore_axis_name="s")

    @functools.partial(
        pl.kernel,
        mesh=mesh,
        out_type=jax.ShapeDtypeStruct((rows * D,), jnp.float32),
        scratch_types=[
            pltpu.VMEM((NE,), jnp.float32),
            pltpu.VMEM((NE,), jnp.float32),
        ],
    )
    def k(x_hbm, pos_hbm, out_hbm, pos_v, x_v):
        wid = lax.axis_index("s") * _NC + lax.axis_index("c")
        row_base = wid * rows_per_w

        def chunk_body(c, _):
            row0 = row_base + c * CH
            pos_row = (row0 % S)
            pltpu.sync_copy(pos_hbm.at[pl.ds(pos_row * D, NE)], pos_v)
            xoff = (b0 * S + row0) * D
            pltpu.sync_copy(x_hbm.at[pl.ds(xoff, NE)], x_v)

            def add_body(j, ___):
                base = j * (_LANES * U)
                for u in range(U):
                    o = base + u * _LANES
                    x_v[pl.ds(o, _LANES)] = (
                        x_v[pl.ds(o, _LANES)] + pos_v[pl.ds(o, _LANES)]
                    )
                return 0

            lax.fori_loop(0, NE // (_LANES * U), add_body, 0)
            pltpu.sync_copy(x_v, out_hbm.at[pl.ds(row0 * D, NE)])
            return 0

        lax.fori_loop(0, NCH, chunk_body, 0)

    return k(xf, posf)


def kernel(x, pos_embedding):
    B, S, D = x.shape
    xf = x.reshape(B * S * D)
    posf = pos_embedding[:S].reshape(S * D)
    return _sc_add_v3(xf, posf, B, S, D).reshape(B, S, D)


# speedup vs baseline: 4.5534x; 4.5534x over previous
import jax
import jax.numpy as jnp
from jax.experimental import pallas as pl
from jax.experimental.pallas import tpu as pltpu


def _add_body(x_ref, pos_ref, out_ref):
    out_ref[...] = x_ref[...] + pos_ref[...][None, :, :]


def kernel(x, pos_embedding):
    B, S, D = x.shape
    BS = 2048
    grid = (S // BS, B)
    return pl.pallas_call(
        _add_body,
        grid=grid,
        in_specs=[
            pl.BlockSpec((1, BS, D), lambda s, b: (b, s, 0)),
            pl.BlockSpec((BS, D), lambda s, b: (s, 0)),
        ],
        out_specs=pl.BlockSpec((1, BS, D), lambda s, b: (b, s, 0)),
        out_shape=jax.ShapeDtypeStruct((B, S, D), x.dtype),
        compiler_params=pltpu.CompilerParams(
            dimension_semantics=("arbitrary", "arbitrary"),
        ),
    )(x, pos_embedding[:S])


# speedup vs baseline: 4.5831x; 1.0065x over previous
import jax
import jax.numpy as jnp
from jax.experimental import pallas as pl
from jax.experimental.pallas import tpu as pltpu

_BS = 2048


def _body(x_hbm, pos_hbm, out_hbm, buf, posbuf, in_sems, out_sems, pos_sems):
    B = 4
    s = pl.program_id(0)
    b = pl.program_id(1)
    k = s * B + b
    nsteps = pl.num_programs(0) * B

    def in_copy(kk):
        ss, bb = kk // B, kk % B
        return pltpu.make_async_copy(
            x_hbm.at[bb, pl.ds(ss * _BS, _BS), :], buf.at[bb], in_sems.at[bb]
        )

    def out_copy(kk):
        ss, bb = kk // B, kk % B
        return pltpu.make_async_copy(
            buf.at[bb], out_hbm.at[bb, pl.ds(ss * _BS, _BS), :], out_sems.at[bb]
        )

    def pos_copy(ss):
        return pltpu.make_async_copy(
            pos_hbm.at[pl.ds(ss * _BS, _BS), :],
            posbuf.at[ss % 2],
            pos_sems.at[ss % 2],
        )

    @pl.when(k == 0)
    def _():
        pos_copy(0).start()
        in_copy(0).start()
        in_copy(1).start()

    @pl.when(k <= nsteps - 3)
    def _():
        @pl.when(k >= 2)
        def _():
            out_copy(k - 2).wait()

        in_copy(k + 2).start()

    @pl.when(jnp.logical_and(b == 2, s + 1 < pl.num_programs(0)))
    def _():
        pos_copy(s + 1).start()

    @pl.when(b == 0)
    def _():
        pos_copy(s).wait()

    in_copy(k).wait()
    buf[b] = buf[b] + posbuf[s % 2]
    out_copy(k).start()

    @pl.when(k == nsteps - 1)
    def _():
        for d in range(4):
            out_copy(nsteps - 4 + d).wait()


def kernel(x, pos_embedding):
    B, S, D = x.shape
    grid = (S // _BS, B)
    return pl.pallas_call(
        _body,
        grid=grid,
        in_specs=[
            pl.BlockSpec(memory_space=pltpu.MemorySpace.HBM),
            pl.BlockSpec(memory_space=pltpu.MemorySpace.HBM),
        ],
        out_specs=pl.BlockSpec(memory_space=pltpu.MemorySpace.HBM),
        out_shape=jax.ShapeDtypeStruct((B, S, D), x.dtype),
        scratch_shapes=[
            pltpu.VMEM((B, _BS, D), jnp.float32),
            pltpu.VMEM((2, _BS, D), jnp.float32),
            pltpu.SemaphoreType.DMA((B,)),
            pltpu.SemaphoreType.DMA((B,)),
            pltpu.SemaphoreType.DMA((2,)),
        ],
        compiler_params=pltpu.CompilerParams(
            dimension_semantics=("arbitrary", "arbitrary"),
        ),
    )(x, pos_embedding[:S])
